# baseline (speedup 1.0000x reference)
import functools

import jax
import jax.numpy as jnp
from jax.experimental import pallas as pl
from jax.experimental.pallas import tpu as pltpu

H, W = 28, 28
KH, KW = 3, 3
OH, OW = H - KH + 1, W - KW + 1
C_OUT = 32
OC = OW * C_OUT
RK = KH * W
N_CLASSES = 10
BN = 2048


def _net_kernel(x_ref, wr_ref, bc_ref, wl_ref, bl_ref, out_ref, xs):
    bn = out_ref.shape[1]
    xs[...] = x_ref[...].reshape(H * W, bn).astype(jnp.bfloat16)
    acc = jnp.zeros((N_CLASSES, bn), jnp.float32)
    for oi in range(OH):
        pat = xs[oi * W:oi * W + RK, :]
        a = jnp.dot(wr_ref[...], pat,
                    preferred_element_type=jnp.float32)
        ab = jnp.maximum(a + bc_ref[...], 0.0).astype(jnp.bfloat16)
        acc = acc + jnp.dot(wl_ref[oi], ab,
                            preferred_element_type=jnp.float32)
    out_ref[...] = acc + bl_ref[...]


@functools.partial(jax.jit, static_argnames=("bn",))
def _forward(x, w_conv, b_conv, w_lin, b_lin, *, bn=BN):
    n = x.shape[0]
    bn = min(bn, max(128, pl.cdiv(n, 128) * 128))
    n_tiles = pl.cdiv(n, bn)
    n_pad = n_tiles * bn
    if n_pad == n:
        xt = jnp.transpose(x, (2, 3, 1, 0)).reshape(H * W, n // 128, 128)
    else:
        xt = jnp.pad(jnp.transpose(x, (2, 3, 1, 0)).reshape(H * W, n),
                     ((0, 0), (0, n_pad - n))).reshape(H * W, n_pad // 128, 128)

    wt = jnp.transpose(w_conv[:, 0, :, :], (1, 2, 0))
    eye = jnp.stack([jnp.eye(W, OW, k=-dj, dtype=w_conv.dtype)
                     for dj in range(KW)])
    wr = (jnp.einsum("jko,djc->ocdk", eye, wt)
             .reshape(OC, RK).astype(jnp.bfloat16))
    bc = jnp.tile(b_conv.astype(jnp.float32), OW).reshape(OC, 1)

    wl = (w_lin.reshape(N_CLASSES, C_OUT, OH, OW)
               .transpose(2, 0, 3, 1)
               .reshape(OH, N_CLASSES, OC)).astype(jnp.bfloat16)
    bl = b_lin.reshape(N_CLASSES, 1).astype(jnp.float32)

    out = pl.pallas_call(
        _net_kernel,
        out_shape=jax.ShapeDtypeStruct((N_CLASSES, n_pad), jnp.float32),
        grid=(n_tiles,),
        in_specs=[
            pl.BlockSpec((H * W, bn // 128, 128), lambda i: (0, i, 0)),
            pl.BlockSpec((OC, RK), lambda i: (0, 0)),
            pl.BlockSpec((OC, 1), lambda i: (0, 0)),
            pl.BlockSpec((OH, N_CLASSES, OC), lambda i: (0, 0, 0)),
            pl.BlockSpec((N_CLASSES, 1), lambda i: (0, 0)),
        ],
        out_specs=pl.BlockSpec((N_CLASSES, bn), lambda i: (0, i)),
        scratch_shapes=[pltpu.VMEM((H * W, bn), jnp.bfloat16)],
        compiler_params=pltpu.CompilerParams(
            dimension_semantics=("parallel",),
            vmem_limit_bytes=64 << 20),
    )(xt, wr, bc, wl, bl)
    return out[:, :n].T


def kernel(x, w_conv, b_conv, w_lin, b_lin):
    return _forward(x, w_conv, b_conv, w_lin, b_lin)


# speedup vs baseline: 1.1440x; 1.1440x over previous
import functools

import jax
import jax.numpy as jnp
from jax.experimental import pallas as pl
from jax.experimental.pallas import tpu as pltpu

H, W = 28, 28
KH, KW = 3, 3
OH, OW = H - KH + 1, W - KW + 1
C_OUT = 32
OC = OW * C_OUT
RK = KH * W
N_CLASSES = 10
BN = 1024


def _net_kernel(x_ref, wr_ref, bc_ref, wl_ref, bl_ref, out_ref, xs):
    bn = out_ref.shape[1]
    xs[...] = x_ref[...].reshape(H * W, bn).astype(jnp.bfloat16)
    acc = jnp.zeros((N_CLASSES, bn), jnp.float32)
    for oi in range(OH):
        pat = xs[oi * W:oi * W + RK, :]
        a = jnp.dot(wr_ref[...], pat,
                    preferred_element_type=jnp.float32)
        ab = jnp.maximum(a + bc_ref[...], 0.0).astype(jnp.bfloat16)
        acc = acc + jnp.dot(wl_ref[oi], ab,
                            preferred_element_type=jnp.float32)
    out_ref[...] = acc + bl_ref[...]


@functools.partial(jax.jit, static_argnames=("bn",))
def _forward(x, w_conv, b_conv, w_lin, b_lin, *, bn=BN):
    n = x.shape[0]
    bn = min(bn, max(128, pl.cdiv(n, 128) * 128))
    n_tiles = pl.cdiv(n, bn)
    n_pad = n_tiles * bn
    if n_pad == n:
        xt = jnp.transpose(x, (2, 3, 1, 0)).reshape(H * W, n // 128, 128)
    else:
        xt = jnp.pad(jnp.transpose(x, (2, 3, 1, 0)).reshape(H * W, n),
                     ((0, 0), (0, n_pad - n))).reshape(H * W, n_pad // 128, 128)

    wt = jnp.transpose(w_conv[:, 0, :, :], (1, 2, 0))
    eye = jnp.stack([jnp.eye(W, OW, k=-dj, dtype=w_conv.dtype)
                     for dj in range(KW)])
    wr = (jnp.einsum("jko,djc->ocdk", eye, wt)
             .reshape(OC, RK).astype(jnp.bfloat16))
    bc = jnp.tile(b_conv.astype(jnp.float32), OW).reshape(OC, 1)

    wl = (w_lin.reshape(N_CLASSES, C_OUT, OH, OW)
               .transpose(2, 0, 3, 1)
               .reshape(OH, N_CLASSES, OC)).astype(jnp.bfloat16)
    bl = b_lin.reshape(N_CLASSES, 1).astype(jnp.float32)

    out = pl.pallas_call(
        _net_kernel,
        out_shape=jax.ShapeDtypeStruct((N_CLASSES, n_pad), jnp.float32),
        grid=(n_tiles,),
        in_specs=[
            pl.BlockSpec((H * W, bn // 128, 128), lambda i: (0, i, 0)),
            pl.BlockSpec((OC, RK), lambda i: (0, 0)),
            pl.BlockSpec((OC, 1), lambda i: (0, 0)),
            pl.BlockSpec((OH, N_CLASSES, OC), lambda i: (0, 0, 0)),
            pl.BlockSpec((N_CLASSES, 1), lambda i: (0, 0)),
        ],
        out_specs=pl.BlockSpec((N_CLASSES, bn), lambda i: (0, i)),
        scratch_shapes=[pltpu.VMEM((H * W, bn), jnp.bfloat16)],
        compiler_params=pltpu.CompilerParams(
            dimension_semantics=("parallel",),
            vmem_limit_bytes=64 << 20),
    )(xt, wr, bc, wl, bl)
    return out[:, :n].T


def kernel(x, w_conv, b_conv, w_lin, b_lin):
    return _forward(x, w_conv, b_conv, w_lin, b_lin)


# speedup vs baseline: 1.1565x; 1.0109x over previous
import functools

import jax
import jax.numpy as jnp
from jax.experimental import pallas as pl
from jax.experimental.pallas import tpu as pltpu

H, W = 28, 28
KH, KW = 3, 3
OH, OW = H - KH + 1, W - KW + 1
C_OUT = 32
OC = OW * C_OUT
RK = KH * W
N_CLASSES = 10
BN = 1024


def _net_kernel(x_ref, wr_ref, bc_ref, wl_ref, bl_ref, out_ref, xs):
    bn = out_ref.shape[1]
    xs[...] = x_ref[...].reshape(H * W, bn).astype(jnp.bfloat16)
    acc = jnp.zeros((N_CLASSES, bn), jnp.float32)
    for k in range(OH // 2):
        pat = xs[k * 2 * W:k * 2 * W + W + RK, :]
        a = jnp.dot(wr_ref[...], pat,
                    preferred_element_type=jnp.float32)
        ab = jnp.maximum(a + bc_ref[...], 0.0).astype(jnp.bfloat16)
        acc = acc + jnp.dot(wl_ref[k], ab,
                            preferred_element_type=jnp.float32)
    out_ref[...] = acc + bl_ref[...]


@functools.partial(jax.jit, static_argnames=("bn",))
def _forward(x, w_conv, b_conv, w_lin, b_lin, *, bn=BN):
    n = x.shape[0]
    bn = min(bn, max(128, pl.cdiv(n, 128) * 128))
    n_tiles = pl.cdiv(n, bn)
    n_pad = n_tiles * bn
    if n_pad == n:
        xt = jnp.transpose(x, (2, 3, 1, 0)).reshape(H * W, n // 128, 128)
    else:
        xt = jnp.pad(jnp.transpose(x, (2, 3, 1, 0)).reshape(H * W, n),
                     ((0, 0), (0, n_pad - n))).reshape(H * W, n_pad // 128, 128)

    wt = jnp.transpose(w_conv[:, 0, :, :], (1, 2, 0))
    eye = jnp.stack([jnp.eye(W, OW, k=-dj, dtype=w_conv.dtype)
                     for dj in range(KW)])
    wr1 = jnp.einsum("jko,djc->ocdk", eye, wt).reshape(OC, RK)
    wr = jnp.zeros((2, OC, W + RK), wr1.dtype)
    wr = wr.at[0, :, :RK].set(wr1).at[1, :, W:W + RK].set(wr1)
    wr = wr.reshape(2 * OC, W + RK).astype(jnp.bfloat16)
    bc = jnp.tile(b_conv.astype(jnp.float32), 2 * OW).reshape(2 * OC, 1)

    wl = (w_lin.reshape(N_CLASSES, C_OUT, OH, OW)
               .transpose(2, 0, 3, 1)
               .reshape(OH // 2, 2, N_CLASSES, OC)
               .transpose(0, 2, 1, 3)
               .reshape(OH // 2, N_CLASSES, 2 * OC)).astype(jnp.bfloat16)
    bl = b_lin.reshape(N_CLASSES, 1).astype(jnp.float32)

    out = pl.pallas_call(
        _net_kernel,
        out_shape=jax.ShapeDtypeStruct((N_CLASSES, n_pad), jnp.float32),
        grid=(n_tiles,),
        in_specs=[
            pl.BlockSpec((H * W, bn // 128, 128), lambda i: (0, i, 0)),
            pl.BlockSpec((2 * OC, W + RK), lambda i: (0, 0)),
            pl.BlockSpec((2 * OC, 1), lambda i: (0, 0)),
            pl.BlockSpec((OH // 2, N_CLASSES, 2 * OC), lambda i: (0, 0, 0)),
            pl.BlockSpec((N_CLASSES, 1), lambda i: (0, 0)),
        ],
        out_specs=pl.BlockSpec((N_CLASSES, bn), lambda i: (0, i)),
        scratch_shapes=[pltpu.VMEM((H * W, bn), jnp.bfloat16)],
        compiler_params=pltpu.CompilerParams(
            dimension_semantics=("parallel",),
            vmem_limit_bytes=64 << 20),
    )(xt, wr, bc, wl, bl)
    return out[:, :n].T


def kernel(x, w_conv, b_conv, w_lin, b_lin):
    return _forward(x, w_conv, b_conv, w_lin, b_lin)
